# Initial kernel scaffold; baseline (speedup 1.0000x reference)
import jax
import jax.numpy as jnp
from jax.experimental import pallas as pl


def kernel(logits, targets):
    raise NotImplementedError("write your pallas kernel here")


# speedup vs baseline: 1.4117x; 1.4117x over previous
import jax
import jax.numpy as jnp
from jax.experimental import pallas as pl

_THRESHOLD = 0.95
_ALPHA = 0.1
_N = 1000
_B = 16384
_BLK = 256
_G = _B // _BLK
_LO = _ALPHA / _N
_HI = 1.0 - _ALPHA + _ALPHA / _N


def _body(x_ref, t_ref, out_ref, mask_ref):
    x = x_ref[...]
    m = jnp.max(x, axis=1, keepdims=True)
    s = jnp.sum(jnp.exp(x - m), axis=1, keepdims=True)
    conf = 1.0 / s
    gt = conf > _THRESHOLD
    cols = jax.lax.broadcasted_iota(jnp.int32, (_BLK, _N), 1)
    amax = jnp.min(jnp.where(x == m, cols, _N), axis=1, keepdims=True)
    tgt = t_ref[0, 0, :].reshape(_BLK, 1)
    hot = jnp.where(gt, amax, tgt)
    out_ref[...] = jnp.where(cols == hot, _HI, _LO)
    mask_ref[0, 0, :] = gt.reshape(_BLK).astype(jnp.float32)


def kernel(logits, targets):
    tg = targets.reshape(_G, 1, _BLK).astype(jnp.int32)
    out, mask = pl.pallas_call(
        _body,
        grid=(_G,),
        in_specs=[
            pl.BlockSpec((_BLK, _N), lambda i: (i, 0)),
            pl.BlockSpec((1, 1, _BLK), lambda i: (i, 0, 0)),
        ],
        out_specs=[
            pl.BlockSpec((_BLK, _N), lambda i: (i, 0)),
            pl.BlockSpec((1, 1, _BLK), lambda i: (i, 0, 0)),
        ],
        out_shape=[
            jax.ShapeDtypeStruct((_B, _N), jnp.float32),
            jax.ShapeDtypeStruct((_G, 1, _BLK), jnp.float32),
        ],
    )(logits, tg)
    return out, mask.reshape(_B)


# speedup vs baseline: 1.5719x; 1.1134x over previous
import jax
import jax.numpy as jnp
from jax.experimental import pallas as pl

_THRESHOLD = 0.95
_ALPHA = 0.1
_N = 1000
_B = 16384
_BLK = 512
_G = _B // _BLK
_LO = _ALPHA / _N
_HI = 1.0 - _ALPHA + _ALPHA / _N


def _body(x_ref, t_ref, out_ref, mask_ref):
    x = x_ref[...]
    m = jnp.max(x, axis=1, keepdims=True)
    s = jnp.sum(jnp.exp(x - m), axis=1, keepdims=True)
    conf = 1.0 / s
    gt = conf > _THRESHOLD
    cols = jax.lax.broadcasted_iota(jnp.int32, (_BLK, _N), 1)
    amax = jnp.min(jnp.where(x == m, cols, _N), axis=1, keepdims=True)
    tgt = t_ref[0, 0, :].reshape(_BLK, 1)
    hot = jnp.where(gt, amax, tgt)
    out_ref[...] = jnp.where(cols == hot, _HI, _LO)
    mask_ref[0, 0, :] = gt.reshape(_BLK).astype(jnp.float32)


def kernel(logits, targets):
    tg = targets.reshape(_G, 1, _BLK).astype(jnp.int32)
    out, mask = pl.pallas_call(
        _body,
        grid=(_G,),
        in_specs=[
            pl.BlockSpec((_BLK, _N), lambda i: (i, 0)),
            pl.BlockSpec((1, 1, _BLK), lambda i: (i, 0, 0)),
        ],
        out_specs=[
            pl.BlockSpec((_BLK, _N), lambda i: (i, 0)),
            pl.BlockSpec((1, 1, _BLK), lambda i: (i, 0, 0)),
        ],
        out_shape=[
            jax.ShapeDtypeStruct((_B, _N), jnp.float32),
            jax.ShapeDtypeStruct((_G, 1, _BLK), jnp.float32),
        ],
    )(logits, tg)
    return out, mask.reshape(_B)


# speedup vs baseline: 1.6589x; 1.0554x over previous
import jax
import jax.numpy as jnp
from jax.experimental import pallas as pl

_THRESHOLD = 0.95
_ALPHA = 0.1
_N = 1000
_B = 16384
_BLK = 1024
_G = _B // _BLK
_LO = _ALPHA / _N
_HI = 1.0 - _ALPHA + _ALPHA / _N


def _body(x_ref, t_ref, out_ref, mask_ref):
    x = x_ref[...]
    m = jnp.max(x, axis=1, keepdims=True)
    s = jnp.sum(jnp.exp(x - m), axis=1, keepdims=True)
    conf = 1.0 / s
    gt = conf > _THRESHOLD
    cols = jax.lax.broadcasted_iota(jnp.int32, (_BLK, _N), 1)
    amax = jnp.min(jnp.where(x == m, cols, _N), axis=1, keepdims=True)
    tgt = t_ref[0, 0, :].reshape(_BLK, 1)
    hot = jnp.where(gt, amax, tgt)
    out_ref[...] = jnp.where(cols == hot, _HI, _LO)
    mask_ref[0, 0, :] = gt.reshape(_BLK).astype(jnp.float32)


def kernel(logits, targets):
    tg = targets.reshape(_G, 1, _BLK).astype(jnp.int32)
    out, mask = pl.pallas_call(
        _body,
        grid=(_G,),
        in_specs=[
            pl.BlockSpec((_BLK, _N), lambda i: (i, 0)),
            pl.BlockSpec((1, 1, _BLK), lambda i: (i, 0, 0)),
        ],
        out_specs=[
            pl.BlockSpec((_BLK, _N), lambda i: (i, 0)),
            pl.BlockSpec((1, 1, _BLK), lambda i: (i, 0, 0)),
        ],
        out_shape=[
            jax.ShapeDtypeStruct((_B, _N), jnp.float32),
            jax.ShapeDtypeStruct((_G, 1, _BLK), jnp.float32),
        ],
    )(logits, tg)
    return out, mask.reshape(_B)


# speedup vs baseline: 1.6878x; 1.0174x over previous
import jax
import jax.numpy as jnp
from jax.experimental import pallas as pl

_THRESHOLD = 0.95
_ALPHA = 0.1
_N = 1000
_B = 16384
_BLK = 2048
_G = _B // _BLK
_LO = _ALPHA / _N
_HI = 1.0 - _ALPHA + _ALPHA / _N


def _body(x_ref, t_ref, out_ref, mask_ref):
    x = x_ref[...]
    m = jnp.max(x, axis=1, keepdims=True)
    s = jnp.sum(jnp.exp(x - m), axis=1, keepdims=True)
    conf = 1.0 / s
    gt = conf > _THRESHOLD
    cols = jax.lax.broadcasted_iota(jnp.int32, (_BLK, _N), 1)
    amax = jnp.min(jnp.where(x == m, cols, _N), axis=1, keepdims=True)
    tgt = t_ref[0, 0, :].reshape(_BLK, 1)
    hot = jnp.where(gt, amax, tgt)
    out_ref[...] = jnp.where(cols == hot, _HI, _LO)
    mask_ref[0, 0, :] = gt.reshape(_BLK).astype(jnp.float32)


def kernel(logits, targets):
    tg = targets.reshape(_G, 1, _BLK).astype(jnp.int32)
    out, mask = pl.pallas_call(
        _body,
        grid=(_G,),
        in_specs=[
            pl.BlockSpec((_BLK, _N), lambda i: (i, 0)),
            pl.BlockSpec((1, 1, _BLK), lambda i: (i, 0, 0)),
        ],
        out_specs=[
            pl.BlockSpec((_BLK, _N), lambda i: (i, 0)),
            pl.BlockSpec((1, 1, _BLK), lambda i: (i, 0, 0)),
        ],
        out_shape=[
            jax.ShapeDtypeStruct((_B, _N), jnp.float32),
            jax.ShapeDtypeStruct((_G, 1, _BLK), jnp.float32),
        ],
    )(logits, tg)
    return out, mask.reshape(_B)
